# baseline (speedup 1.0000x reference)
import functools

import jax
import jax.numpy as jnp
from jax import lax
from jax.experimental import pallas as pl
from jax.experimental.pallas import tpu as pltpu
from jax.experimental.pallas import tpu_sc as plsc

N = 10000
NP = 10240
E = 320000
H = 128
T = 3
L = 3

NUM_CORES = 2
NUM_SUBCORES = 16
NUM_WORKERS = NUM_CORES * NUM_SUBCORES
GC = 48
GSLOTS = 6
SSLICES = 3
SG = 2 * GC
GROUPS = 112
DHALF = 64
CHUNKS = 2 * GROUPS
EPW_P = GROUPS * SG
EP = NUM_WORKERS * EPW_P
ROWS_PER_TILE = NP // NUM_SUBCORES
FLUSH_CHUNK = 128


def _make_sc_agg(num_towers: int):
    mesh = plsc.VectorSubcoreMesh(core_axis_name="c", subcore_axis_name="s")

    def body(h_hbm, src_hbm, dst_hbm, zeros_hbm, out_hbm,
             src_blk, d0, d1, d2, ring, acc, *sems):
        c = lax.axis_index("c")
        s = lax.axis_index("s")
        wid = c * NUM_SUBCORES + s
        dbufs = (d0, d1, d2)
        gsems = sems[:GSLOTS]
        dsems = sems[GSLOTS:GSLOTS + SSLICES]
        ssems = sems[GSLOTS + SSLICES:]

        pltpu.sync_copy(src_hbm.at[wid], src_blk)

        for t in range(num_towers):
            pltpu.sync_copy(zeros_hbm, ring.at[pl.ds(0, FLUSH_CHUNK)])
            r0 = s * ROWS_PER_TILE
            for k in range(ROWS_PER_TILE // FLUSH_CHUNK):
                pltpu.sync_copy(
                    ring.at[pl.ds(0, FLUSH_CHUNK)],
                    acc.at[pl.ds(r0 + k * FLUSH_CHUNK, FLUSH_CHUNK)])
            plsc.subcore_barrier()

            table = h_hbm.at[t]

            def fire_d(g, sl):
                pltpu.async_copy(dst_hbm.at[wid, g], dbufs[sl], dsems[sl])

            def wait_d(g, sl):
                pltpu.make_async_copy(dst_hbm.at[wid, g], dbufs[sl],
                                      dsems[sl]).wait()

            def fire_g(ch, q):
                off = pl.multiple_of(ch * GC, GC)
                pltpu.async_copy(table.at[src_blk.at[pl.ds(off, GC)]],
                                 ring.at[pl.ds(q * GC, GC)], gsems[q])

            def wait_g(ch, q):
                off = pl.multiple_of(ch * GC, GC)
                pltpu.make_async_copy(table.at[src_blk.at[pl.ds(off, GC)]],
                                      ring.at[pl.ds(q * GC, GC)],
                                      gsems[q]).wait()

            def fire_s(sl):
                pltpu.async_copy(ring.at[pl.ds(2 * sl * GC, SG)],
                                 acc.at[dbufs[sl]], ssems[sl], add=True)

            def wait_s(sl):
                pltpu.make_async_copy(ring.at[pl.ds(2 * sl * GC, SG)],
                                      acc.at[dbufs[sl]], ssems[sl]).wait()

            def group_body(g, sl, wait_prev, pf_d, fire_next):
                q0, q1 = 2 * sl, 2 * sl + 1
                wait_g(2 * g, q0)
                wait_g(2 * g + 1, q1)
                wait_d(g, sl)
                fire_s(sl)
                sln = (sl + 2) % SSLICES
                if wait_prev:
                    wait_s(sln)
                if pf_d:
                    fire_d(g + 2, sln)
                if fire_next:
                    fire_g(2 * g + 4, 2 * sln)
                    fire_g(2 * g + 5, 2 * sln + 1)

            fire_d(0, 0)
            fire_d(1, 1)
            for q in range(4):
                fire_g(q, q)
            group_body(0, 0, False, True, True)

            def triple(p, carry):
                g = 3 * p + 1
                group_body(g, 1, True, True, True)
                group_body(g + 1, 2, True, True, True)
                group_body(g + 2, 0, True, True, True)
                return carry

            lax.fori_loop(0, 36, triple, 0)

            group_body(109, 1, True, True, True)
            group_body(110, 2, True, False, False)
            group_body(111, 0, True, False, False)
            wait_s(0)

            plsc.subcore_barrier()

            for k in range(ROWS_PER_TILE // FLUSH_CHUNK):
                off = r0 + k * FLUSH_CHUNK
                pltpu.sync_copy(acc.at[pl.ds(off, FLUSH_CHUNK)],
                                ring.at[pl.ds(0, FLUSH_CHUNK)])
                pltpu.sync_copy(ring.at[pl.ds(0, FLUSH_CHUNK)],
                                out_hbm.at[c, t, pl.ds(off, FLUSH_CHUNK)])
            plsc.subcore_barrier()

    return pl.kernel(
        body,
        out_type=jax.ShapeDtypeStruct((NUM_CORES, num_towers, NP, H),
                                      jnp.float32),
        mesh=mesh,
        scratch_types=(
            [pltpu.VMEM((EPW_P,), jnp.int32)]
            + [pltpu.VMEM((SG,), jnp.int32) for _ in range(SSLICES)]
            + [pltpu.VMEM((GSLOTS * GC, H), jnp.float32),
               pltpu.VMEM_SHARED((NP, H), jnp.float32)]
            + [pltpu.SemaphoreType.DMA] * (GSLOTS + 2 * SSLICES)
        ),
    )


_sc_agg_1 = _make_sc_agg(1)
_sc_agg_3 = _make_sc_agg(T)

BN = 1024
GRID = NP // BN


def _mm(a, w):
    return lax.dot_general(a, w, (((1,), (0,)), ((), ())),
                           preferred_element_type=jnp.float32,
                           precision=lax.Precision.HIGHEST)


def _mlp_first_body(scale_ref, x_ref, aggp_ref, w1_ref, b1_ref, w2_ref,
                    b2_ref, out_ref):
    agg = aggp_ref[0] + aggp_ref[1]
    x = x_ref[...]
    for t in range(T):
        u = scale_ref[t] * x + agg
        v = jnp.maximum(_mm(u, w1_ref[t]) + b1_ref[t], 0.0)
        w = jnp.maximum(_mm(v, w2_ref[t]) + b2_ref[t], 0.0)
        out_ref[t] = w


def _mlp_mid_body(scale_ref, h_ref, aggp_ref, w1_ref, b1_ref, w2_ref,
                  b2_ref, out_ref):
    for t in range(T):
        u = scale_ref[t] * h_ref[t] + (aggp_ref[0, t] + aggp_ref[1, t])
        v = jnp.maximum(_mm(u, w1_ref[t]) + b1_ref[t], 0.0)
        w = jnp.maximum(_mm(v, w2_ref[t]) + b2_ref[t], 0.0)
        out_ref[t] = w


_W_SPEC = pl.BlockSpec((T, H, H), lambda i: (0, 0, 0))
_B_SPEC = pl.BlockSpec((T, H), lambda i: (0, 0))
_H3_SPEC = pl.BlockSpec((T, BN, H), lambda i: (0, i, 0))

_mlp_first = pl.pallas_call(
    _mlp_first_body,
    grid=(GRID,),
    in_specs=[
        pl.BlockSpec(memory_space=pltpu.SMEM),
        pl.BlockSpec((BN, H), lambda i: (i, 0)),
        pl.BlockSpec((NUM_CORES, BN, H), lambda i: (0, i, 0)),
        _W_SPEC, _B_SPEC, _W_SPEC, _B_SPEC,
    ],
    out_specs=_H3_SPEC,
    out_shape=jax.ShapeDtypeStruct((T, NP, H), jnp.float32),
)

_mlp_mid = pl.pallas_call(
    _mlp_mid_body,
    grid=(GRID,),
    in_specs=[
        pl.BlockSpec(memory_space=pltpu.SMEM),
        _H3_SPEC,
        pl.BlockSpec((NUM_CORES, T, BN, H), lambda i: (0, 0, i, 0)),
        _W_SPEC, _B_SPEC, _W_SPEC, _B_SPEC,
    ],
    out_specs=_H3_SPEC,
    out_shape=jax.ShapeDtypeStruct((T, NP, H), jnp.float32),
)


def kernel(x, edge_index, W1, b1, W2, b2, eps):
    src = jnp.concatenate(
        [edge_index[0], jnp.zeros((EP - E,), jnp.int32)]
    ).reshape(NUM_WORKERS, EPW_P)
    dst = jnp.concatenate(
        [edge_index[1], jnp.full((EP - E,), N, jnp.int32)]
    ).reshape(NUM_WORKERS, GROUPS, SG)
    scale = 1.0 + eps
    zeros = jnp.zeros((FLUSH_CHUNK, H), jnp.float32)
    xp = jnp.pad(x, ((0, NP - N), (0, 0)))

    aggp0 = _sc_agg_1(xp[None], src, dst, zeros)
    h = _mlp_first(scale[:, 0], xp, aggp0[:, 0],
                   W1[:, 0], b1[:, 0], W2[:, 0], b2[:, 0])
    for l in range(1, L):
        aggp = _sc_agg_3(h, src, dst, zeros)
        h = _mlp_mid(scale[:, l], h, aggp,
                     W1[:, l], b1[:, l], W2[:, l], b2[:, l])
    return jnp.transpose(h[:, :N], (1, 0, 2))


# speedup vs baseline: 1.0099x; 1.0099x over previous
import functools

import jax
import jax.numpy as jnp
from jax import lax
from jax.experimental import pallas as pl
from jax.experimental.pallas import tpu as pltpu
from jax.experimental.pallas import tpu_sc as plsc

N = 10000
NP = 10240
E = 320000
H = 128
T = 3
L = 3

NUM_CORES = 2
NUM_SUBCORES = 16
NUM_WORKERS = NUM_CORES * NUM_SUBCORES
GC = 48
GSLOTS = 6
SSLICES = 3
SG = 2 * GC
GROUPS = 112
DHALF = 64
CHUNKS = 2 * GROUPS
EPW_P = GROUPS * SG
EP = NUM_WORKERS * EPW_P
ROWS_PER_TILE = NP // NUM_SUBCORES
FLUSH_CHUNK = 128


def _make_sc_agg(num_towers: int):
    mesh = plsc.VectorSubcoreMesh(core_axis_name="c", subcore_axis_name="s")

    def body(h_hbm, src_hbm, dst_hbm, zeros_hbm, out_hbm,
             src_blk, d0, d1, d2, ring, acc, *sems):
        c = lax.axis_index("c")
        s = lax.axis_index("s")
        wid = c * NUM_SUBCORES + s
        dbufs = (d0, d1, d2)
        gsems = sems[:GSLOTS]
        dsems = sems[GSLOTS:GSLOTS + SSLICES]
        ssems = sems[GSLOTS + SSLICES:]

        pltpu.sync_copy(src_hbm.at[wid], src_blk)

        for t in range(num_towers):
            pltpu.sync_copy(zeros_hbm, ring.at[pl.ds(0, FLUSH_CHUNK)])
            r0 = s * ROWS_PER_TILE
            for k in range(ROWS_PER_TILE // FLUSH_CHUNK):
                pltpu.sync_copy(
                    ring.at[pl.ds(0, FLUSH_CHUNK)],
                    acc.at[pl.ds(r0 + k * FLUSH_CHUNK, FLUSH_CHUNK)])
            plsc.subcore_barrier()

            table = h_hbm.at[t]

            def fire_d(g, sl):
                pltpu.async_copy(dst_hbm.at[wid, g], dbufs[sl], dsems[sl])

            def wait_d(g, sl):
                pltpu.make_async_copy(dst_hbm.at[wid, g], dbufs[sl],
                                      dsems[sl]).wait()

            def fire_g(ch, q):
                off = pl.multiple_of(ch * GC, GC)
                pltpu.async_copy(table.at[src_blk.at[pl.ds(off, GC)]],
                                 ring.at[pl.ds(q * GC, GC)], gsems[q])

            def wait_g(ch, q):
                off = pl.multiple_of(ch * GC, GC)
                pltpu.make_async_copy(table.at[src_blk.at[pl.ds(off, GC)]],
                                      ring.at[pl.ds(q * GC, GC)],
                                      gsems[q]).wait()

            def fire_s(sl):
                pltpu.async_copy(ring.at[pl.ds(2 * sl * GC, SG)],
                                 acc.at[dbufs[sl]], ssems[sl], add=True)

            def wait_s(sl):
                pltpu.make_async_copy(ring.at[pl.ds(2 * sl * GC, SG)],
                                      acc.at[dbufs[sl]], ssems[sl]).wait()

            def group_body(g, sl, wait_prev, pf_d, fire_next):
                q0, q1 = 2 * sl, 2 * sl + 1
                wait_g(2 * g, q0)
                wait_g(2 * g + 1, q1)
                wait_d(g, sl)
                fire_s(sl)
                sln = (sl + 2) % SSLICES
                if wait_prev:
                    wait_s(sln)
                if pf_d:
                    fire_d(g + 2, sln)
                if fire_next:
                    fire_g(2 * g + 4, 2 * sln)
                    fire_g(2 * g + 5, 2 * sln + 1)

            fire_d(0, 0)
            fire_d(1, 1)
            for q in range(4):
                fire_g(q, q)
            group_body(0, 0, False, True, True)

            def triple(p, carry):
                g = 3 * p + 1
                group_body(g, 1, True, True, True)
                group_body(g + 1, 2, True, True, True)
                group_body(g + 2, 0, True, True, True)
                return carry

            lax.fori_loop(0, 36, triple, 0)

            group_body(109, 1, True, True, True)
            group_body(110, 2, True, False, False)
            group_body(111, 0, True, False, False)
            wait_s(0)

            plsc.subcore_barrier()

            for k in range(ROWS_PER_TILE // FLUSH_CHUNK):
                off = r0 + k * FLUSH_CHUNK
                pltpu.sync_copy(acc.at[pl.ds(off, FLUSH_CHUNK)],
                                ring.at[pl.ds(0, FLUSH_CHUNK)])
                pltpu.sync_copy(ring.at[pl.ds(0, FLUSH_CHUNK)],
                                out_hbm.at[c, t, pl.ds(off, FLUSH_CHUNK)])
            plsc.subcore_barrier()

    return pl.kernel(
        body,
        out_type=jax.ShapeDtypeStruct((NUM_CORES, num_towers, NP, H),
                                      jnp.float32),
        mesh=mesh,
        scratch_types=(
            [pltpu.VMEM((EPW_P,), jnp.int32)]
            + [pltpu.VMEM((SG,), jnp.int32) for _ in range(SSLICES)]
            + [pltpu.VMEM((GSLOTS * GC, H), jnp.float32),
               pltpu.VMEM_SHARED((NP, H), jnp.float32)]
            + [pltpu.SemaphoreType.DMA] * (GSLOTS + 2 * SSLICES)
        ),
    )


_sc_agg_1 = _make_sc_agg(1)

BN = 1024
GRID = NP // BN


def _mm(a, w):
    return lax.dot_general(a, w, (((1,), (0,)), ((), ())),
                           preferred_element_type=jnp.float32,
                           precision=lax.Precision.HIGHEST)


def _mlp_first_body(scale_ref, x_ref, aggp_ref, w1_ref, b1_ref, w2_ref,
                    b2_ref, out_ref):
    agg = aggp_ref[0] + aggp_ref[1]
    x = x_ref[...]
    for t in range(T):
        u = scale_ref[t] * x + agg
        v = jnp.maximum(_mm(u, w1_ref[t]) + b1_ref[t], 0.0)
        w = jnp.maximum(_mm(v, w2_ref[t]) + b2_ref[t], 0.0)
        out_ref[t] = w


def _mlp_mid_body(scale_ref, h_ref, ap0_ref, ap1_ref, ap2_ref, w1_ref,
                  b1_ref, w2_ref, b2_ref, out_ref):
    aps = (ap0_ref, ap1_ref, ap2_ref)
    for t in range(T):
        u = scale_ref[t] * h_ref[t] + (aps[t][0, 0] + aps[t][1, 0])
        v = jnp.maximum(_mm(u, w1_ref[t]) + b1_ref[t], 0.0)
        w = jnp.maximum(_mm(v, w2_ref[t]) + b2_ref[t], 0.0)
        out_ref[t] = w


_W_SPEC = pl.BlockSpec((T, H, H), lambda i: (0, 0, 0))
_B_SPEC = pl.BlockSpec((T, H), lambda i: (0, 0))
_H3_SPEC = pl.BlockSpec((T, BN, H), lambda i: (0, i, 0))

_mlp_first = pl.pallas_call(
    _mlp_first_body,
    grid=(GRID,),
    in_specs=[
        pl.BlockSpec(memory_space=pltpu.SMEM),
        pl.BlockSpec((BN, H), lambda i: (i, 0)),
        pl.BlockSpec((NUM_CORES, BN, H), lambda i: (0, i, 0)),
        _W_SPEC, _B_SPEC, _W_SPEC, _B_SPEC,
    ],
    out_specs=_H3_SPEC,
    out_shape=jax.ShapeDtypeStruct((T, NP, H), jnp.float32),
)

_AP_SPEC = pl.BlockSpec((NUM_CORES, 1, BN, H), lambda i: (0, 0, i, 0))

_mlp_mid = pl.pallas_call(
    _mlp_mid_body,
    grid=(GRID,),
    in_specs=[
        pl.BlockSpec(memory_space=pltpu.SMEM),
        _H3_SPEC,
        _AP_SPEC, _AP_SPEC, _AP_SPEC,
        _W_SPEC, _B_SPEC, _W_SPEC, _B_SPEC,
    ],
    out_specs=_H3_SPEC,
    out_shape=jax.ShapeDtypeStruct((T, NP, H), jnp.float32),
)


def kernel(x, edge_index, W1, b1, W2, b2, eps):
    src = jnp.concatenate(
        [edge_index[0], jnp.zeros((EP - E,), jnp.int32)]
    ).reshape(NUM_WORKERS, EPW_P)
    dst = jnp.concatenate(
        [edge_index[1], jnp.full((EP - E,), N, jnp.int32)]
    ).reshape(NUM_WORKERS, GROUPS, SG)
    scale = 1.0 + eps
    zeros = jnp.zeros((FLUSH_CHUNK, H), jnp.float32)
    xp = jnp.pad(x, ((0, NP - N), (0, 0)))

    aggp0 = _sc_agg_1(xp[None], src, dst, zeros)
    h = _mlp_first(scale[:, 0], xp, aggp0[:, 0],
                   W1[:, 0], b1[:, 0], W2[:, 0], b2[:, 0])
    for l in range(1, L):
        aps = [_sc_agg_1(h[t][None], src, dst, zeros) for t in range(T)]
        h = _mlp_mid(scale[:, l], h, aps[0], aps[1], aps[2],
                     W1[:, l], b1[:, l], W2[:, l], b2[:, l])
    return jnp.transpose(h[:, :N], (1, 0, 2))


# speedup vs baseline: 1.2669x; 1.2545x over previous
import functools

import jax
import jax.numpy as jnp
from jax import lax
from jax.experimental import pallas as pl
from jax.experimental.pallas import tpu as pltpu
from jax.experimental.pallas import tpu_sc as plsc

N = 10000
NP = 10240
E = 320000
H = 128
T = 3
L = 3

NUM_CORES = 2
NUM_SUBCORES = 16
NUM_WORKERS = NUM_CORES * NUM_SUBCORES
GC = 48
NBUF = 5
CHUNKS = 220
EPW_P = CHUNKS * GC
EP = NUM_WORKERS * EPW_P
ROWS_PER_TILE = NP // NUM_SUBCORES
ZTAIL = ROWS_PER_TILE - (ROWS_PER_TILE // GC) * GC


def _make_sc_agg(num_towers: int):
    mesh = plsc.VectorSubcoreMesh(core_axis_name="c", subcore_axis_name="s")

    def body(h_hbm, src_hbm, dst_hbm, zeros_hbm, out_hbm,
             src_blk, d0, d1, d2, d3, d4, r0_, r1_, r2_, r3_, r4_, acc,
             *sems):
        c = lax.axis_index("c")
        s = lax.axis_index("s")
        wid = c * NUM_SUBCORES + s
        dbufs = (d0, d1, d2, d3, d4)
        rbufs = (r0_, r1_, r2_, r3_, r4_)
        gsems = sems[:NBUF]
        dsems = sems[NBUF:2 * NBUF]
        ssems = sems[2 * NBUF:3 * NBUF]

        pltpu.sync_copy(src_hbm.at[wid], src_blk)

        for t in range(num_towers):
            pltpu.sync_copy(zeros_hbm, r0_)
            base = s * ROWS_PER_TILE
            for k in range(ROWS_PER_TILE // GC):
                pltpu.sync_copy(
                    r0_, acc.at[pl.ds(base + k * GC, GC)])
            pltpu.sync_copy(
                r0_.at[pl.ds(0, ZTAIL)],
                acc.at[pl.ds(base + (ROWS_PER_TILE // GC) * GC, ZTAIL)])
            plsc.subcore_barrier()

            table = h_hbm.at[t]

            def fire_d(ch, b):
                pltpu.async_copy(dst_hbm.at[wid, ch], dbufs[b], dsems[b])

            def wait_d(ch, b):
                pltpu.make_async_copy(dst_hbm.at[wid, ch], dbufs[b],
                                      dsems[b]).wait()

            def fire_g(ch, b):
                off = pl.multiple_of(ch * GC, GC)
                pltpu.async_copy(table.at[src_blk.at[pl.ds(off, GC)]],
                                 rbufs[b], gsems[b])

            def wait_g(ch, b):
                off = pl.multiple_of(ch * GC, GC)
                pltpu.make_async_copy(table.at[src_blk.at[pl.ds(off, GC)]],
                                      rbufs[b], gsems[b]).wait()

            def fire_s(b):
                pltpu.async_copy(rbufs[b], acc.at[dbufs[b]], ssems[b],
                                 add=True)

            def wait_s(b):
                pltpu.make_async_copy(rbufs[b], acc.at[dbufs[b]],
                                      ssems[b]).wait()

            def chunk_body(ch, b, wait_prev, fire_next):
                wait_g(ch, b)
                wait_d(ch, b)
                fire_s(b)
                bn = (b + 4) % NBUF
                if wait_prev:
                    wait_s(bn)
                if fire_next:
                    fire_d(ch + 4, bn)
                    fire_g(ch + 4, bn)

            for b in range(NBUF):
                fire_d(b, b)
                fire_g(b, b)
            chunk_body(0, 0, False, False)

            def block(p, carry):
                cbase = 5 * p + 1
                for j in range(5):
                    chunk_body(cbase + j, (1 + j) % NBUF, True, True)
                return carry

            lax.fori_loop(0, 43, block, 0)

            for ch in range(216, 220):
                chunk_body(ch, ch % NBUF, True, False)
            wait_s(219 % NBUF)

            plsc.subcore_barrier()

            for k in range(ROWS_PER_TILE // GC):
                off = base + k * GC
                pltpu.sync_copy(acc.at[pl.ds(off, GC)], r0_)
                pltpu.sync_copy(r0_, out_hbm.at[c, t, pl.ds(off, GC)])
            toff = base + (ROWS_PER_TILE // GC) * GC
            pltpu.sync_copy(acc.at[pl.ds(toff, ZTAIL)],
                            r0_.at[pl.ds(0, ZTAIL)])
            pltpu.sync_copy(r0_.at[pl.ds(0, ZTAIL)],
                            out_hbm.at[c, t, pl.ds(toff, ZTAIL)])
            plsc.subcore_barrier()

    return pl.kernel(
        body,
        out_type=jax.ShapeDtypeStruct((NUM_CORES, num_towers, NP, H),
                                      jnp.float32),
        mesh=mesh,
        scratch_types=(
            [pltpu.VMEM((EPW_P,), jnp.int32)]
            + [pltpu.VMEM((GC,), jnp.int32) for _ in range(NBUF)]
            + [pltpu.VMEM((GC, H), jnp.float32) for _ in range(NBUF)]
            + [pltpu.VMEM_SHARED((NP, H), jnp.float32)]
            + [pltpu.SemaphoreType.DMA] * (3 * NBUF)
        ),
    )


_sc_agg_1 = _make_sc_agg(1)

BN = 1024
GRID = NP // BN


def _mm(a, w):
    return lax.dot_general(a, w, (((1,), (0,)), ((), ())),
                           preferred_element_type=jnp.float32,
                           precision=lax.Precision.HIGHEST)


def _mlp_first_body(scale_ref, x_ref, aggp_ref, w1_ref, b1_ref, w2_ref,
                    b2_ref, out_ref):
    agg = aggp_ref[0] + aggp_ref[1]
    x = x_ref[...]
    for t in range(T):
        u = scale_ref[t] * x + agg
        v = jnp.maximum(_mm(u, w1_ref[t]) + b1_ref[t], 0.0)
        w = jnp.maximum(_mm(v, w2_ref[t]) + b2_ref[t], 0.0)
        out_ref[t] = w


def _mlp_mid_body(scale_ref, h_ref, ap0_ref, ap1_ref, ap2_ref, w1_ref,
                  b1_ref, w2_ref, b2_ref, out_ref):
    aps = (ap0_ref, ap1_ref, ap2_ref)
    for t in range(T):
        u = scale_ref[t] * h_ref[t] + (aps[t][0, 0] + aps[t][1, 0])
        v = jnp.maximum(_mm(u, w1_ref[t]) + b1_ref[t], 0.0)
        w = jnp.maximum(_mm(v, w2_ref[t]) + b2_ref[t], 0.0)
        out_ref[t] = w


_W_SPEC = pl.BlockSpec((T, H, H), lambda i: (0, 0, 0))
_B_SPEC = pl.BlockSpec((T, H), lambda i: (0, 0))
_H3_SPEC = pl.BlockSpec((T, BN, H), lambda i: (0, i, 0))

_mlp_first = pl.pallas_call(
    _mlp_first_body,
    grid=(GRID,),
    in_specs=[
        pl.BlockSpec(memory_space=pltpu.SMEM),
        pl.BlockSpec((BN, H), lambda i: (i, 0)),
        pl.BlockSpec((NUM_CORES, BN, H), lambda i: (0, i, 0)),
        _W_SPEC, _B_SPEC, _W_SPEC, _B_SPEC,
    ],
    out_specs=_H3_SPEC,
    out_shape=jax.ShapeDtypeStruct((T, NP, H), jnp.float32),
)

_AP_SPEC = pl.BlockSpec((NUM_CORES, 1, BN, H), lambda i: (0, 0, i, 0))

_mlp_mid = pl.pallas_call(
    _mlp_mid_body,
    grid=(GRID,),
    in_specs=[
        pl.BlockSpec(memory_space=pltpu.SMEM),
        _H3_SPEC,
        _AP_SPEC, _AP_SPEC, _AP_SPEC,
        _W_SPEC, _B_SPEC, _W_SPEC, _B_SPEC,
    ],
    out_specs=_H3_SPEC,
    out_shape=jax.ShapeDtypeStruct((T, NP, H), jnp.float32),
)


def kernel(x, edge_index, W1, b1, W2, b2, eps):
    src = jnp.concatenate(
        [edge_index[0], jnp.zeros((EP - E,), jnp.int32)]
    ).reshape(NUM_WORKERS, EPW_P)
    dst = jnp.concatenate(
        [edge_index[1], jnp.full((EP - E,), N, jnp.int32)]
    ).reshape(NUM_WORKERS, CHUNKS, GC)
    scale = 1.0 + eps
    zeros = jnp.zeros((GC, H), jnp.float32)
    xp = jnp.pad(x, ((0, NP - N), (0, 0)))

    aggp0 = _sc_agg_1(xp[None], src, dst, zeros)
    h = _mlp_first(scale[:, 0], xp, aggp0[:, 0],
                   W1[:, 0], b1[:, 0], W2[:, 0], b2[:, 0])
    for l in range(1, L):
        aps = [_sc_agg_1(h[t][None], src, dst, zeros) for t in range(T)]
        h = _mlp_mid(scale[:, l], h, aps[0], aps[1], aps[2],
                     W1[:, l], b1[:, l], W2[:, l], b2[:, l])
    return jnp.transpose(h[:, :N], (1, 0, 2))


# speedup vs baseline: 1.3497x; 1.0654x over previous
import functools

import jax
import jax.numpy as jnp
from jax import lax
from jax.experimental import pallas as pl
from jax.experimental.pallas import tpu as pltpu
from jax.experimental.pallas import tpu_sc as plsc

N = 10000
NP = 10240
E = 320000
H = 128
T = 3
L = 3

NUM_CORES = 2
NUM_SUBCORES = 16
NUM_WORKERS = NUM_CORES * NUM_SUBCORES
GC = 48
NBUF = 5
CHUNKS = 220
EPW_P = CHUNKS * GC
EP = NUM_WORKERS * EPW_P
ROWS_PER_TILE = NP // NUM_SUBCORES
ZTAIL = ROWS_PER_TILE - (ROWS_PER_TILE // GC) * GC


def _make_sc_agg(num_towers: int):
    mesh = plsc.VectorSubcoreMesh(core_axis_name="c", subcore_axis_name="s")

    def body(h_hbm, src_hbm, dst_hbm, zeros_hbm, out_hbm,
             src_blk, d0, d1, d2, d3, d4, r0_, r1_, r2_, r3_, r4_, acc,
             *sems):
        c = lax.axis_index("c")
        s = lax.axis_index("s")
        wid = c * NUM_SUBCORES + s
        dbase = wid * EPW_P
        dbufs = (d0, d1, d2, d3, d4)
        rbufs = (r0_, r1_, r2_, r3_, r4_)
        gsems = sems[:NBUF]
        dsems = sems[NBUF:2 * NBUF]
        ssems = sems[2 * NBUF:3 * NBUF]

        pltpu.sync_copy(src_hbm.at[wid], src_blk)

        for t in range(num_towers):
            pltpu.sync_copy(zeros_hbm, r0_)
            base = s * ROWS_PER_TILE
            for k in range(ROWS_PER_TILE // GC):
                pltpu.sync_copy(
                    r0_, acc.at[pl.ds(base + k * GC, GC)])
            pltpu.sync_copy(
                r0_.at[pl.ds(0, ZTAIL)],
                acc.at[pl.ds(base + (ROWS_PER_TILE // GC) * GC, ZTAIL)])
            plsc.subcore_barrier()

            table = h_hbm.at[t]

            def fire_d(ch, b):
                off = pl.multiple_of(ch * GC, GC)
                pltpu.async_copy(dst_hbm.at[pl.ds(dbase + off, GC)],
                                 dbufs[b], dsems[b])

            def wait_d(ch, b):
                off = pl.multiple_of(ch * GC, GC)
                pltpu.make_async_copy(dst_hbm.at[pl.ds(dbase + off, GC)],
                                      dbufs[b], dsems[b]).wait()

            def fire_g(ch, b):
                off = pl.multiple_of(ch * GC, GC)
                pltpu.async_copy(table.at[src_blk.at[pl.ds(off, GC)]],
                                 rbufs[b], gsems[b])

            def wait_g(ch, b):
                off = pl.multiple_of(ch * GC, GC)
                pltpu.make_async_copy(table.at[src_blk.at[pl.ds(off, GC)]],
                                      rbufs[b], gsems[b]).wait()

            def fire_s(b):
                pltpu.async_copy(rbufs[b], acc.at[dbufs[b]], ssems[b],
                                 add=True)

            def wait_s(b):
                pltpu.make_async_copy(rbufs[b], acc.at[dbufs[b]],
                                      ssems[b]).wait()

            def chunk_body(ch, b, wait_prev, fire_next):
                wait_g(ch, b)
                wait_d(ch, b)
                fire_s(b)
                bn = (b + 4) % NBUF
                if wait_prev:
                    wait_s(bn)
                if fire_next:
                    fire_d(ch + 4, bn)
                    fire_g(ch + 4, bn)

            for b in range(NBUF):
                fire_d(b, b)
                fire_g(b, b)
            chunk_body(0, 0, False, False)

            def block(p, carry):
                cbase = 5 * p + 1
                for j in range(5):
                    chunk_body(cbase + j, (1 + j) % NBUF, True, True)
                return carry

            lax.fori_loop(0, 43, block, 0)

            for ch in range(216, 220):
                chunk_body(ch, ch % NBUF, True, False)
            wait_s(219 % NBUF)

            plsc.subcore_barrier()

            for k in range(ROWS_PER_TILE // GC):
                off = base + k * GC
                pltpu.sync_copy(acc.at[pl.ds(off, GC)], r0_)
                pltpu.sync_copy(r0_, out_hbm.at[c, t, pl.ds(off, GC)])
            toff = base + (ROWS_PER_TILE // GC) * GC
            pltpu.sync_copy(acc.at[pl.ds(toff, ZTAIL)],
                            r0_.at[pl.ds(0, ZTAIL)])
            pltpu.sync_copy(r0_.at[pl.ds(0, ZTAIL)],
                            out_hbm.at[c, t, pl.ds(toff, ZTAIL)])
            plsc.subcore_barrier()

    return pl.kernel(
        body,
        out_type=jax.ShapeDtypeStruct((NUM_CORES, num_towers, NP, H),
                                      jnp.float32),
        mesh=mesh,
        scratch_types=(
            [pltpu.VMEM((EPW_P,), jnp.int32)]
            + [pltpu.VMEM((GC,), jnp.int32) for _ in range(NBUF)]
            + [pltpu.VMEM((GC, H), jnp.float32) for _ in range(NBUF)]
            + [pltpu.VMEM_SHARED((NP, H), jnp.float32)]
            + [pltpu.SemaphoreType.DMA] * (3 * NBUF)
        ),
    )


_sc_agg_1 = _make_sc_agg(1)

BN = 1024
GRID = NP // BN


def _mm(a, w):
    return lax.dot_general(a, w, (((1,), (0,)), ((), ())),
                           preferred_element_type=jnp.float32,
                           precision=lax.Precision.HIGHEST)


def _mlp_first_body(scale_ref, x_ref, aggp_ref, w1_ref, b1_ref, w2_ref,
                    b2_ref, out_ref):
    agg = aggp_ref[0] + aggp_ref[1]
    x = x_ref[...]
    for t in range(T):
        u = scale_ref[t] * x + agg
        v = jnp.maximum(_mm(u, w1_ref[t]) + b1_ref[t], 0.0)
        w = jnp.maximum(_mm(v, w2_ref[t]) + b2_ref[t], 0.0)
        out_ref[t] = w


def _mlp_mid_body(scale_ref, h_ref, ap0_ref, ap1_ref, ap2_ref, w1_ref,
                  b1_ref, w2_ref, b2_ref, out_ref):
    aps = (ap0_ref, ap1_ref, ap2_ref)
    for t in range(T):
        u = scale_ref[t] * h_ref[t] + (aps[t][0, 0] + aps[t][1, 0])
        v = jnp.maximum(_mm(u, w1_ref[t]) + b1_ref[t], 0.0)
        w = jnp.maximum(_mm(v, w2_ref[t]) + b2_ref[t], 0.0)
        out_ref[t] = w


_W_SPEC = pl.BlockSpec((T, H, H), lambda i: (0, 0, 0))
_B_SPEC = pl.BlockSpec((T, H), lambda i: (0, 0))
_H3_SPEC = pl.BlockSpec((T, BN, H), lambda i: (0, i, 0))

_mlp_first = pl.pallas_call(
    _mlp_first_body,
    grid=(GRID,),
    in_specs=[
        pl.BlockSpec(memory_space=pltpu.SMEM),
        pl.BlockSpec((BN, H), lambda i: (i, 0)),
        pl.BlockSpec((NUM_CORES, BN, H), lambda i: (0, i, 0)),
        _W_SPEC, _B_SPEC, _W_SPEC, _B_SPEC,
    ],
    out_specs=_H3_SPEC,
    out_shape=jax.ShapeDtypeStruct((T, NP, H), jnp.float32),
)

_AP_SPEC = pl.BlockSpec((NUM_CORES, 1, BN, H), lambda i: (0, 0, i, 0))

_mlp_mid = pl.pallas_call(
    _mlp_mid_body,
    grid=(GRID,),
    in_specs=[
        pl.BlockSpec(memory_space=pltpu.SMEM),
        _H3_SPEC,
        _AP_SPEC, _AP_SPEC, _AP_SPEC,
        _W_SPEC, _B_SPEC, _W_SPEC, _B_SPEC,
    ],
    out_specs=_H3_SPEC,
    out_shape=jax.ShapeDtypeStruct((T, NP, H), jnp.float32),
)


def kernel(x, edge_index, W1, b1, W2, b2, eps):
    src = jnp.concatenate(
        [edge_index[0], jnp.zeros((EP - E,), jnp.int32)]
    ).reshape(NUM_WORKERS, EPW_P)
    dst = jnp.concatenate(
        [edge_index[1], jnp.full((EP - E,), N, jnp.int32)])
    scale = 1.0 + eps
    zeros = jnp.zeros((GC, H), jnp.float32)
    xp = jnp.pad(x, ((0, NP - N), (0, 0)))

    aggp0 = _sc_agg_1(xp[None], src, dst, zeros)
    h = _mlp_first(scale[:, 0], xp, aggp0[:, 0],
                   W1[:, 0], b1[:, 0], W2[:, 0], b2[:, 0])
    for l in range(1, L):
        aps = [_sc_agg_1(h[t][None], src, dst, zeros) for t in range(T)]
        h = _mlp_mid(scale[:, l], h, aps[0], aps[1], aps[2],
                     W1[:, l], b1[:, l], W2[:, l], b2[:, l])
    return jnp.transpose(h[:, :N], (1, 0, 2))


# speedup vs baseline: 7.4017x; 5.4839x over previous
import functools

import jax
import jax.numpy as jnp
from jax import lax
from jax.experimental import pallas as pl
from jax.experimental.pallas import tpu as pltpu
from jax.experimental.pallas import tpu_sc as plsc

N = 10000
NP = 10240
E = 320000
H = 128
T = 3
L = 3

NUM_CORES = 2
NUM_SUBCORES = 16
NUM_WORKERS = NUM_CORES * NUM_SUBCORES
EPW = E // NUM_WORKERS
CHUNK = 128
FULL_CHUNKS = EPW // CHUNK
TAIL = EPW - FULL_CHUNKS * CHUNK
PAIRS = FULL_CHUNKS // 2
ROWS_PER_TILE = NP // NUM_SUBCORES
FLUSH_CHUNK = 128


def _make_sc_agg(num_towers: int):
    mesh = plsc.VectorSubcoreMesh(core_axis_name="c", subcore_axis_name="s")

    def body(h_hbm, src_hbm, dst_hbm, zeros_hbm, out_hbm,
             idx0, idx1, dst0, dst1, idxT, dstT,
             rows0, rows1, rowsT, acc, sem0, sem1):
        c = lax.axis_index("c")
        s = lax.axis_index("s")
        wid = c * NUM_SUBCORES + s
        base = wid * EPW

        for t in range(num_towers):
            pltpu.sync_copy(zeros_hbm, rows1)
            r0 = s * ROWS_PER_TILE
            for k in range(ROWS_PER_TILE // FLUSH_CHUNK):
                pltpu.sync_copy(
                    rows1,
                    acc.at[pl.ds(r0 + k * FLUSH_CHUNK, FLUSH_CHUNK)])
            plsc.subcore_barrier()

            table = h_hbm.at[t]

            def fire(j, idxbuf, rowsbuf, sem):
                pltpu.sync_copy(src_hbm.at[pl.ds(base + j * CHUNK, CHUNK)],
                                idxbuf)
                pltpu.async_copy(table.at[idxbuf], rowsbuf, sem)

            def wait_rows(idxbuf, rowsbuf, sem):
                pltpu.make_async_copy(table.at[idxbuf], rowsbuf, sem).wait()

            def scat(j, dstbuf, rowsbuf):
                pltpu.sync_copy(dst_hbm.at[pl.ds(base + j * CHUNK, CHUNK)],
                                dstbuf)
                pltpu.sync_copy(rowsbuf, acc.at[dstbuf], add=True)

            fire(0, idx0, rows0, sem0)
            fire(1, idx1, rows1, sem1)

            def pair_body(p, carry):
                j0 = 2 * p
                wait_rows(idx0, rows0, sem0)
                scat(j0, dst0, rows0)
                fire(j0 + 2, idx0, rows0, sem0)
                wait_rows(idx1, rows1, sem1)
                scat(j0 + 1, dst1, rows1)
                fire(j0 + 3, idx1, rows1, sem1)
                return carry

            lax.fori_loop(0, PAIRS - 1, pair_body, 0)

            j_last = 2 * (PAIRS - 1)
            wait_rows(idx0, rows0, sem0)
            scat(j_last, dst0, rows0)
            wait_rows(idx1, rows1, sem1)
            scat(j_last + 1, dst1, rows1)

            tb = base + FULL_CHUNKS * CHUNK
            pltpu.sync_copy(src_hbm.at[pl.ds(tb, TAIL)], idxT)
            pltpu.async_copy(table.at[idxT], rowsT, sem0).wait()
            pltpu.sync_copy(dst_hbm.at[pl.ds(tb, TAIL)], dstT)
            pltpu.sync_copy(rowsT, acc.at[dstT], add=True)

            plsc.subcore_barrier()

            for k in range(ROWS_PER_TILE // FLUSH_CHUNK):
                off = r0 + k * FLUSH_CHUNK
                pltpu.sync_copy(acc.at[pl.ds(off, FLUSH_CHUNK)],
                                rows0.at[pl.ds(0, FLUSH_CHUNK)])
                pltpu.sync_copy(rows0.at[pl.ds(0, FLUSH_CHUNK)],
                                out_hbm.at[c, t, pl.ds(off, FLUSH_CHUNK)])
            plsc.subcore_barrier()

    return pl.kernel(
        body,
        out_type=jax.ShapeDtypeStruct((NUM_CORES, num_towers, NP, H),
                                      jnp.float32),
        mesh=mesh,
        scratch_types=[
            pltpu.VMEM((CHUNK,), jnp.int32),
            pltpu.VMEM((CHUNK,), jnp.int32),
            pltpu.VMEM((CHUNK,), jnp.int32),
            pltpu.VMEM((CHUNK,), jnp.int32),
            pltpu.VMEM((TAIL,), jnp.int32),
            pltpu.VMEM((TAIL,), jnp.int32),
            pltpu.VMEM((CHUNK, H), jnp.float32),
            pltpu.VMEM((CHUNK, H), jnp.float32),
            pltpu.VMEM((TAIL, H), jnp.float32),
            pltpu.VMEM_SHARED((NP, H), jnp.float32),
            pltpu.SemaphoreType.DMA,
            pltpu.SemaphoreType.DMA,
        ],
    )


_sc_agg_1 = _make_sc_agg(1)
_sc_agg_3 = _make_sc_agg(T)

BN = 1024
GRID = NP // BN


def _mm(a, w):
    return lax.dot_general(a, w, (((1,), (0,)), ((), ())),
                           preferred_element_type=jnp.float32,
                           precision=lax.Precision.HIGHEST)


def _mlp_first_body(scale_ref, x_ref, aggp_ref, w1_ref, b1_ref, w2_ref,
                    b2_ref, out_ref):
    agg = aggp_ref[0] + aggp_ref[1]
    x = x_ref[...]
    for t in range(T):
        u = scale_ref[t] * x + agg
        v = jnp.maximum(_mm(u, w1_ref[t]) + b1_ref[t], 0.0)
        w = jnp.maximum(_mm(v, w2_ref[t]) + b2_ref[t], 0.0)
        out_ref[t] = w


def _mlp_mid_body(scale_ref, h_ref, aggp_ref, w1_ref, b1_ref, w2_ref,
                  b2_ref, out_ref):
    for t in range(T):
        u = scale_ref[t] * h_ref[t] + (aggp_ref[0, t] + aggp_ref[1, t])
        v = jnp.maximum(_mm(u, w1_ref[t]) + b1_ref[t], 0.0)
        w = jnp.maximum(_mm(v, w2_ref[t]) + b2_ref[t], 0.0)
        out_ref[t] = w


_W_SPEC = pl.BlockSpec((T, H, H), lambda i: (0, 0, 0))
_B_SPEC = pl.BlockSpec((T, H), lambda i: (0, 0))
_H3_SPEC = pl.BlockSpec((T, BN, H), lambda i: (0, i, 0))

_mlp_first = pl.pallas_call(
    _mlp_first_body,
    grid=(GRID,),
    in_specs=[
        pl.BlockSpec(memory_space=pltpu.SMEM),
        pl.BlockSpec((BN, H), lambda i: (i, 0)),
        pl.BlockSpec((NUM_CORES, BN, H), lambda i: (0, i, 0)),
        _W_SPEC, _B_SPEC, _W_SPEC, _B_SPEC,
    ],
    out_specs=_H3_SPEC,
    out_shape=jax.ShapeDtypeStruct((T, NP, H), jnp.float32),
)

_mlp_mid = pl.pallas_call(
    _mlp_mid_body,
    grid=(GRID,),
    in_specs=[
        pl.BlockSpec(memory_space=pltpu.SMEM),
        _H3_SPEC,
        pl.BlockSpec((NUM_CORES, T, BN, H), lambda i: (0, 0, i, 0)),
        _W_SPEC, _B_SPEC, _W_SPEC, _B_SPEC,
    ],
    out_specs=_H3_SPEC,
    out_shape=jax.ShapeDtypeStruct((T, NP, H), jnp.float32),
)


def kernel(x, edge_index, W1, b1, W2, b2, eps):
    src = edge_index[0]
    dst = edge_index[1]
    scale = 1.0 + eps
    zeros = jnp.zeros((CHUNK, H), jnp.float32)
    xp = jnp.pad(x, ((0, NP - N), (0, 0)))

    aggp0 = _sc_agg_1(xp[None], src, dst, zeros)
    h = _mlp_first(scale[:, 0], xp, aggp0[:, 0],
                   W1[:, 0], b1[:, 0], W2[:, 0], b2[:, 0])
    for l in range(1, L):
        aggp = _sc_agg_3(h, src, dst, zeros)
        h = _mlp_mid(scale[:, l], h, aggp,
                     W1[:, l], b1[:, l], W2[:, l], b2[:, l])
    return jnp.transpose(h[:, :N], (1, 0, 2))
